# baseline (speedup 1.0000x reference)
import jax
import jax.numpy as jnp
from jax.experimental import pallas as pl

_K = 3
_BM = 1024
_BE = 4096


def _elu(v):
    return jnp.where(v > 0, v, jnp.exp(jnp.minimum(v, 0.0)) - 1.0)


def _mm_kernel(x_ref, w_ref, o_ref):
    o_ref[...] = jnp.dot(x_ref[...], w_ref[...], preferred_element_type=jnp.float32)


def _elu_mm_kernel(a_ref, b_ref, w_ref, o_ref):
    h = _elu(a_ref[...] + b_ref[...])
    o_ref[...] = jnp.dot(h, w_ref[...], preferred_element_type=jnp.float32)


def _elu_bias_kernel(a_ref, b_ref, o_ref):
    o_ref[...] = _elu(a_ref[...] + b_ref[...])


def _cdist_kernel(zm_ref, z_ref, clm_ref, cl_ref, o_ref):
    zm = zm_ref[...]
    z = z_ref[...]
    d2 = (jnp.sum(zm * zm, axis=1, keepdims=True)
          + jnp.sum(z * z, axis=1)[None, :]
          - 2.0 * jax.lax.dot_general(zm, z, (((1,), (1,)), ((), ())),
                                      preferred_element_type=jnp.float32))
    dist = jnp.sqrt(jnp.maximum(d2, 0.0))
    same = clm_ref[...] == cl_ref[...]
    o_ref[...] = jnp.where(same, dist, jnp.inf)


def _dec_kernel(z1_ref, z2_ref, w1_ref, b1_ref, w2_ref, b2_ref, o_ref):
    h = z1_ref[...] * z2_ref[...]
    h = jnp.maximum(jnp.dot(h, w1_ref[...], preferred_element_type=jnp.float32)
                    + b1_ref[...], 0.0)
    o_ref[...] = jnp.dot(h, w2_ref[...], preferred_element_type=jnp.float32) + b2_ref[...]


def _norm_kernel(z_ref, o_ref):
    z = z_ref[...]
    nrm = jnp.maximum(jnp.sqrt(jnp.sum(z * z, axis=1, keepdims=True)), 1e-12)
    o_ref[...] = z / nrm


def _pallas_mm(x, w):
    n, d = x.shape
    h = w.shape[1]
    return pl.pallas_call(
        _mm_kernel,
        grid=(pl.cdiv(n, _BM),),
        in_specs=[pl.BlockSpec((_BM, d), lambda i: (i, 0)),
                  pl.BlockSpec((d, h), lambda i: (0, 0))],
        out_specs=pl.BlockSpec((_BM, h), lambda i: (i, 0)),
        out_shape=jax.ShapeDtypeStruct((n, h), jnp.float32),
    )(x, w)


def _pallas_elu_mm(a, b, w):
    n, d = a.shape
    h = w.shape[1]
    return pl.pallas_call(
        _elu_mm_kernel,
        grid=(pl.cdiv(n, _BM),),
        in_specs=[pl.BlockSpec((_BM, d), lambda i: (i, 0)),
                  pl.BlockSpec((1, d), lambda i: (0, 0)),
                  pl.BlockSpec((d, h), lambda i: (0, 0))],
        out_specs=pl.BlockSpec((_BM, h), lambda i: (i, 0)),
        out_shape=jax.ShapeDtypeStruct((n, h), jnp.float32),
    )(a, b[None, :], w)


def _pallas_elu_bias(a, b):
    n, h = a.shape
    return pl.pallas_call(
        _elu_bias_kernel,
        grid=(pl.cdiv(n, _BM),),
        in_specs=[pl.BlockSpec((_BM, h), lambda i: (i, 0)),
                  pl.BlockSpec((1, h), lambda i: (0, 0))],
        out_specs=pl.BlockSpec((_BM, h), lambda i: (i, 0)),
        out_shape=jax.ShapeDtypeStruct((n, h), jnp.float32),
    )(a, b[None, :])


def _pallas_cdist(zm, z, clm, cl):
    m, d = zm.shape
    n = z.shape[0]
    return pl.pallas_call(
        _cdist_kernel,
        grid=(pl.cdiv(n, _BM),),
        in_specs=[pl.BlockSpec((m, d), lambda i: (0, 0)),
                  pl.BlockSpec((_BM, d), lambda i: (i, 0)),
                  pl.BlockSpec((m, 1), lambda i: (0, 0)),
                  pl.BlockSpec((1, _BM), lambda i: (0, i))],
        out_specs=pl.BlockSpec((m, _BM), lambda i: (0, i)),
        out_shape=jax.ShapeDtypeStruct((m, n), jnp.float32),
    )(zm, z, clm, cl)


def _pallas_decoder(z1, z2, w1, b1, w2, b2):
    e, h = z1.shape
    w2p = jnp.zeros((h, 128), jnp.float32).at[:, 0].set(w2[:, 0])
    b2p = jnp.zeros((1, 128), jnp.float32).at[0, 0].set(b2[0])
    out = pl.pallas_call(
        _dec_kernel,
        grid=(pl.cdiv(e, _BE),),
        in_specs=[pl.BlockSpec((_BE, h), lambda i: (i, 0)),
                  pl.BlockSpec((_BE, h), lambda i: (i, 0)),
                  pl.BlockSpec((h, h), lambda i: (0, 0)),
                  pl.BlockSpec((1, h), lambda i: (0, 0)),
                  pl.BlockSpec((h, 128), lambda i: (0, 0)),
                  pl.BlockSpec((1, 128), lambda i: (0, 0))],
        out_specs=pl.BlockSpec((_BE, 128), lambda i: (i, 0)),
        out_shape=jax.ShapeDtypeStruct((e, 128), jnp.float32),
    )(z1, z2, w1, b1[None, :], w2p, b2p)
    return out[:, :1]


def _pallas_rownorm(z):
    n, h = z.shape
    return pl.pallas_call(
        _norm_kernel,
        grid=(pl.cdiv(n, _BM),),
        in_specs=[pl.BlockSpec((_BM, h), lambda i: (i, 0))],
        out_specs=pl.BlockSpec((_BM, h), lambda i: (i, 0)),
        out_shape=jax.ShapeDtypeStruct((n, h), jnp.float32),
    )(z)


def kernel(x, edge_index, masked_edges, neg_edges, cluster_labels, m_indices,
           W1, b1, W2, b2, DW1, Db1, DW2, Db2):
    n = x.shape[0]
    h = W1.shape[1]

    loop = jnp.arange(n, dtype=edge_index.dtype)
    src = jnp.concatenate([edge_index[0], loop])
    dst = jnp.concatenate([edge_index[1], loop])
    deg = jnp.zeros((n,), jnp.float32).at[dst].add(1.0)
    dinv = jax.lax.rsqrt(jnp.maximum(deg, 1e-12))
    enorm = dinv[src] * dinv[dst]

    xw = _pallas_mm(x, W1)
    agg1 = jnp.zeros((n, h), jnp.float32).at[dst].add(
        jnp.take(xw, src, axis=0) * enorm[:, None])
    hw = _pallas_elu_mm(agg1, b1, W2)
    agg2 = jnp.zeros((n, h), jnp.float32).at[dst].add(
        jnp.take(hw, src, axis=0) * enorm[:, None])
    Z = _pallas_elu_bias(agg2, b2)

    zm = jnp.take(Z, m_indices, axis=0)
    clm = jnp.take(cluster_labels, m_indices).astype(jnp.int32)[:, None]
    cl = cluster_labels.astype(jnp.int32)[None, :]
    dist = _pallas_cdist(zm, Z, clm, cl)
    _, idx = jax.lax.top_k(-dist, _K + 1)
    nbr = jnp.mean(jnp.take(Z, idx[:, 1:], axis=0), axis=1)
    Z = Z.at[m_indices].add(nbr)

    pos = _pallas_decoder(jnp.take(Z, masked_edges[0], axis=0),
                          jnp.take(Z, masked_edges[1], axis=0),
                          DW1, Db1, DW2, Db2)
    neg = _pallas_decoder(jnp.take(Z, neg_edges[0], axis=0),
                          jnp.take(Z, neg_edges[1], axis=0),
                          DW1, Db1, DW2, Db2)

    Zn = _pallas_rownorm(Z)
    return (Zn, pos, neg)
